# baseline (speedup 1.0000x reference)
import jax
import jax.numpy as jnp
from jax import lax
from jax.experimental import pallas as pl
from jax.experimental.pallas import tpu as pltpu

_STD = (((1,), (0,)), ((), ()))
_C01 = (((0,), (1,)), ((), ()))
_BB = 2


def _dot(a, b):
    return lax.dot_general(a, b, _STD, preferred_element_type=jnp.float32)


def _body(adj_ref, x_ref, ws1_ref, wn1_ref, b1_ref, ws2_ref, wn2_ref, b2_ref,
          y0_ref, y1_ref, at_s, adjt_s, ub0_s, ub1_s, ub2_s, bias_s):
    ldim = ws1_ref.shape[1]
    cl = x_ref.shape[2]
    cdim = cl // ldim
    bb = x_ref.shape[0]

    @pl.when(pl.program_id(0) == 0)
    def _init():
        adj = adj_ref[...]
        mask = (adj != 0.0).astype(jnp.float32)
        deg_row = jnp.sum(mask, axis=0, keepdims=True)
        inv_row = jnp.where(deg_row > 0.0, 1.0 / jnp.clip(deg_row, 1.0, None), 0.0)
        at_s[...] = jnp.transpose(mask * inv_row).astype(jnp.bfloat16)
        adjt_s[...] = jnp.transpose(adj * (1.0 / 3.5)).astype(jnp.bfloat16)
        ws1 = ws1_ref[...]
        wn1 = wn1_ref[...]
        ws2 = ws2_ref[...]
        wn2 = wn2_ref[...]
        u0 = lax.dot_general(ws1, ws2, _C01)
        u1 = lax.dot_general(wn1, ws2, _C01) + lax.dot_general(ws1, wn2, _C01)
        u2 = lax.dot_general(wn1, wn2, _C01)
        rblk = lax.broadcasted_iota(jnp.int32, (cl, cl), 0) // ldim
        cblk = lax.broadcasted_iota(jnp.int32, (cl, cl), 1) // ldim
        dmask = (rblk == cblk).astype(jnp.float32)
        ub0_s[...] = (jnp.tile(u0 * 3.5, (cdim, cdim)) * dmask).astype(jnp.bfloat16)
        ub1_s[...] = (jnp.tile(u1 * 3.5, (cdim, cdim)) * dmask).astype(jnp.bfloat16)
        ub2_s[...] = (jnp.tile(u2 * 3.5, (cdim, cdim)) * dmask).astype(jnp.bfloat16)
        b1v = b1_ref[...]
        b2v = b2_ref[...]
        c1 = lax.dot_general(b1v, ws2, (((1,), (1,)), ((), ()))) + b2v
        c2 = lax.dot_general(b1v, wn2, (((1,), (1,)), ((), ())))
        e_row = jnp.where(deg_row > 0.0, 1.0, 0.0)
        ecol = jnp.transpose(e_row)
        bias_s[...] = jnp.tile(3.5 * (c1 + ecol * c2), (1, cdim))

    at = at_s[...]
    for j in range(bb):
        f = x_ref[j]
        s1 = _dot(at, f)
        s1b = s1.astype(jnp.bfloat16)
        s2b = _dot(at, s1b).astype(jnp.bfloat16)
        h = (_dot(f, ub0_s[...])
             + _dot(s1b, ub1_s[...])
             + _dot(s2b, ub2_s[...])
             + bias_s[...])
        hb = h.astype(jnp.bfloat16)
        y0_ref[j] = hb
        y1_ref[j] = _dot(adjt_s[...], hb).astype(jnp.bfloat16)


_CHUNKS = 4


def kernel(x, adj, W_self1, W_neigh1, b1, W_self2, W_neigh2, b2):
    B, C, N, L = x.shape
    b1r = b1.reshape(1, L)
    b2r = b2.reshape(1, L)
    full = lambda shape: pl.BlockSpec(shape, lambda i: (0,) * len(shape))

    def run_chunk(xh):
        Bh = xh.shape[0]
        xt = jnp.transpose(xh, (0, 2, 1, 3)).reshape(Bh, N, C * L)
        xt = xt.astype(jnp.bfloat16)
        y0, y1 = pl.pallas_call(
            _body,
            grid=(Bh // _BB,),
            in_specs=[
                full((N, N)),
                pl.BlockSpec((_BB, N, C * L), lambda i: (i, 0, 0)),
                full((L, L)), full((L, L)), full((1, L)),
                full((L, L)), full((L, L)), full((1, L)),
            ],
            out_specs=[
                pl.BlockSpec((_BB, N, C * L), lambda i: (i, 0, 0)),
                pl.BlockSpec((_BB, N, C * L), lambda i: (i, 0, 0)),
            ],
            out_shape=[
                jax.ShapeDtypeStruct((Bh, N, C * L), jnp.bfloat16),
                jax.ShapeDtypeStruct((Bh, N, C * L), jnp.bfloat16),
            ],
            scratch_shapes=[
                pltpu.VMEM((N, N), jnp.bfloat16),
                pltpu.VMEM((N, N), jnp.bfloat16),
                pltpu.VMEM((C * L, C * L), jnp.bfloat16),
                pltpu.VMEM((C * L, C * L), jnp.bfloat16),
                pltpu.VMEM((C * L, C * L), jnp.bfloat16),
                pltpu.VMEM((N, C * L), jnp.float32),
            ],
        )(adj, xt, W_self1, W_neigh1, b1r, W_self2, W_neigh2, b2r)
        y0t = jnp.transpose(y0.reshape(Bh, N, C, L), (0, 2, 1, 3))
        y1t = jnp.transpose(y1.reshape(Bh, N, C, L), (0, 2, 1, 3))
        out = jnp.stack([y0t, y1t], axis=2).reshape(Bh, 2 * C, N, L)
        return out.astype(jnp.float32)

    step = B // _CHUNKS
    outs = [run_chunk(x[i * step:(i + 1) * step]) for i in range(_CHUNKS)]
    return jnp.concatenate(outs, axis=0)


# speedup vs baseline: 1.1451x; 1.1451x over previous
import jax
import jax.numpy as jnp
from jax import lax
from jax.experimental import pallas as pl
from jax.experimental.pallas import tpu as pltpu

_STD = (((1,), (0,)), ((), ()))
_C01 = (((0,), (1,)), ((), ()))
_BB = 2


def _dot(a, b):
    return lax.dot_general(a, b, _STD, preferred_element_type=jnp.float32)


def _body(adj_ref, x_ref, ws1_ref, wn1_ref, b1_ref, ws2_ref, wn2_ref, b2_ref,
          y0_ref, y1_ref, at_s, adjt_s, ub0_s, ub1_s, ub2_s, bias_s):
    ldim = ws1_ref.shape[1]
    cl = x_ref.shape[2]
    cdim = cl // ldim
    bb = x_ref.shape[0]

    @pl.when(pl.program_id(0) == 0)
    def _init():
        adj = adj_ref[...]
        mask = (adj != 0.0).astype(jnp.float32)
        deg_row = jnp.sum(mask, axis=0, keepdims=True)
        inv_row = jnp.where(deg_row > 0.0, 1.0 / jnp.clip(deg_row, 1.0, None), 0.0)
        at_s[...] = jnp.transpose(mask * inv_row).astype(jnp.bfloat16)
        adjt_s[...] = jnp.transpose(adj * (1.0 / 3.5)).astype(jnp.bfloat16)
        ws1 = ws1_ref[...]
        wn1 = wn1_ref[...]
        ws2 = ws2_ref[...]
        wn2 = wn2_ref[...]
        u0 = lax.dot_general(ws1, ws2, _C01)
        u1 = lax.dot_general(wn1, ws2, _C01) + lax.dot_general(ws1, wn2, _C01)
        u2 = lax.dot_general(wn1, wn2, _C01)
        rblk = lax.broadcasted_iota(jnp.int32, (cl, cl), 0) // ldim
        cblk = lax.broadcasted_iota(jnp.int32, (cl, cl), 1) // ldim
        dmask = (rblk == cblk).astype(jnp.float32)
        ub0_s[...] = (jnp.tile(u0 * 3.5, (cdim, cdim)) * dmask).astype(jnp.bfloat16)
        ub1_s[...] = (jnp.tile(u1 * 3.5, (cdim, cdim)) * dmask).astype(jnp.bfloat16)
        ub2_s[...] = (jnp.tile(u2 * 3.5, (cdim, cdim)) * dmask).astype(jnp.bfloat16)
        b1v = b1_ref[...]
        b2v = b2_ref[...]
        c1 = lax.dot_general(b1v, ws2, (((1,), (1,)), ((), ()))) + b2v
        c2 = lax.dot_general(b1v, wn2, (((1,), (1,)), ((), ())))
        e_row = jnp.where(deg_row > 0.0, 1.0, 0.0)
        ecol = jnp.transpose(e_row)
        bias_s[...] = jnp.tile(3.5 * (c1 + ecol * c2), (1, cdim))

    at = at_s[...]
    for j in range(bb):
        f = x_ref[j]
        s1 = _dot(at, f)
        s1b = s1.astype(jnp.bfloat16)
        s2b = _dot(at, s1b).astype(jnp.bfloat16)
        h = (_dot(f, ub0_s[...])
             + _dot(s1b, ub1_s[...])
             + _dot(s2b, ub2_s[...])
             + bias_s[...])
        hb = h.astype(jnp.bfloat16)
        y0_ref[j] = hb
        y1_ref[j] = _dot(adjt_s[...], hb).astype(jnp.bfloat16)


def kernel(x, adj, W_self1, W_neigh1, b1, W_self2, W_neigh2, b2):
    B, C, N, L = x.shape
    xt = jnp.transpose(x, (0, 2, 1, 3)).reshape(B, N, C * L).astype(jnp.bfloat16)
    b1r = b1.reshape(1, L)
    b2r = b2.reshape(1, L)

    full = lambda shape: pl.BlockSpec(shape, lambda i: (0,) * len(shape))
    y0, y1 = pl.pallas_call(
        _body,
        grid=(B // _BB,),
        in_specs=[
            full((N, N)),
            pl.BlockSpec((_BB, N, C * L), lambda i: (i, 0, 0)),
            full((L, L)), full((L, L)), full((1, L)),
            full((L, L)), full((L, L)), full((1, L)),
        ],
        out_specs=[
            pl.BlockSpec((_BB, N, C * L), lambda i: (i, 0, 0)),
            pl.BlockSpec((_BB, N, C * L), lambda i: (i, 0, 0)),
        ],
        out_shape=[
            jax.ShapeDtypeStruct((B, N, C * L), jnp.bfloat16),
            jax.ShapeDtypeStruct((B, N, C * L), jnp.bfloat16),
        ],
        scratch_shapes=[
            pltpu.VMEM((N, N), jnp.bfloat16),
            pltpu.VMEM((N, N), jnp.bfloat16),
            pltpu.VMEM((C * L, C * L), jnp.bfloat16),
            pltpu.VMEM((C * L, C * L), jnp.bfloat16),
            pltpu.VMEM((C * L, C * L), jnp.bfloat16),
            pltpu.VMEM((N, C * L), jnp.float32),
        ],
    )(adj, xt, W_self1, W_neigh1, b1r, W_self2, W_neigh2, b2r)

    y0t = jnp.transpose(y0.reshape(B, N, C, L), (0, 2, 1, 3))
    y1t = jnp.transpose(y1.reshape(B, N, C, L), (0, 2, 1, 3))
    return jnp.stack([y0t, y1t], axis=2).reshape(B, 2 * C, N, L).astype(jnp.float32)


# speedup vs baseline: 1.2664x; 1.1060x over previous
import jax
import jax.numpy as jnp
from jax import lax
from jax.experimental import pallas as pl
from jax.experimental.pallas import tpu as pltpu

_STD = (((1,), (0,)), ((), ()))
_C01 = (((0,), (1,)), ((), ()))
_BB = 2


def _dot(a, b):
    return lax.dot_general(a, b, _STD, preferred_element_type=jnp.float32)


def _body(adj_ref, x_ref, ws1_ref, wn1_ref, b1_ref, ws2_ref, wn2_ref, b2_ref,
          y0_ref, at_s, adjt_s, ub0_s, ub1_s, ub2_s, bias_s):
    ldim = ws1_ref.shape[1]
    cl = x_ref.shape[2]
    cdim = cl // ldim
    bb = x_ref.shape[0]

    @pl.when(pl.program_id(0) == 0)
    def _init():
        adj = adj_ref[...]
        mask = (adj != 0.0).astype(jnp.float32)
        deg_row = jnp.sum(mask, axis=0, keepdims=True)
        inv_row = jnp.where(deg_row > 0.0, 1.0 / jnp.clip(deg_row, 1.0, None), 0.0)
        at_s[...] = jnp.transpose(mask * inv_row).astype(jnp.bfloat16)
        adjt_s[...] = jnp.transpose(adj * (1.0 / 3.5)).astype(jnp.bfloat16)
        ws1 = ws1_ref[...]
        wn1 = wn1_ref[...]
        ws2 = ws2_ref[...]
        wn2 = wn2_ref[...]
        u0 = lax.dot_general(ws1, ws2, _C01)
        u1 = lax.dot_general(wn1, ws2, _C01) + lax.dot_general(ws1, wn2, _C01)
        u2 = lax.dot_general(wn1, wn2, _C01)
        rblk = lax.broadcasted_iota(jnp.int32, (cl, cl), 0) // ldim
        cblk = lax.broadcasted_iota(jnp.int32, (cl, cl), 1) // ldim
        dmask = (rblk == cblk).astype(jnp.float32)
        ub0_s[...] = (jnp.tile(u0 * 3.5, (cdim, cdim)) * dmask).astype(jnp.bfloat16)
        ub1_s[...] = (jnp.tile(u1 * 3.5, (cdim, cdim)) * dmask).astype(jnp.bfloat16)
        ub2_s[...] = (jnp.tile(u2 * 3.5, (cdim, cdim)) * dmask).astype(jnp.bfloat16)
        b1v = b1_ref[...]
        b2v = b2_ref[...]
        c1 = lax.dot_general(b1v, ws2, (((1,), (1,)), ((), ()))) + b2v
        c2 = lax.dot_general(b1v, wn2, (((1,), (1,)), ((), ())))
        e_row = jnp.where(deg_row > 0.0, 1.0, 0.0)
        ecol = jnp.transpose(e_row)
        bias_s[...] = jnp.tile(3.5 * (c1 + ecol * c2), (1, cdim))

    at = at_s[...]
    for j in range(bb):
        f = x_ref[j]
        s1 = _dot(at, f)
        s1b = s1.astype(jnp.bfloat16)
        s2b = _dot(at, s1b).astype(jnp.bfloat16)
        h = (_dot(f, ub0_s[...])
             + _dot(s1b, ub1_s[...])
             + _dot(s2b, ub2_s[...])
             + bias_s[...])
        hb = h.astype(jnp.bfloat16)
        y0_ref[0, j] = hb
        y0_ref[1, j] = _dot(adjt_s[...], hb).astype(jnp.bfloat16)


def kernel(x, adj, W_self1, W_neigh1, b1, W_self2, W_neigh2, b2):
    B, C, N, L = x.shape
    xt = jnp.transpose(x, (0, 2, 1, 3)).reshape(B, N, C * L).astype(jnp.bfloat16)
    b1r = b1.reshape(1, L)
    b2r = b2.reshape(1, L)

    full = lambda shape: pl.BlockSpec(shape, lambda i: (0,) * len(shape))
    y = pl.pallas_call(
        _body,
        grid=(B // _BB,),
        in_specs=[
            full((N, N)),
            pl.BlockSpec((_BB, N, C * L), lambda i: (i, 0, 0)),
            full((L, L)), full((L, L)), full((1, L)),
            full((L, L)), full((L, L)), full((1, L)),
        ],
        out_specs=[
            pl.BlockSpec((2, _BB, N, C * L), lambda i: (0, i, 0, 0)),
        ],
        out_shape=[
            jax.ShapeDtypeStruct((2, B, N, C * L), jnp.bfloat16),
        ],
        scratch_shapes=[
            pltpu.VMEM((N, N), jnp.bfloat16),
            pltpu.VMEM((N, N), jnp.bfloat16),
            pltpu.VMEM((C * L, C * L), jnp.bfloat16),
            pltpu.VMEM((C * L, C * L), jnp.bfloat16),
            pltpu.VMEM((C * L, C * L), jnp.bfloat16),
            pltpu.VMEM((N, C * L), jnp.float32),
        ],
    )(adj, xt, W_self1, W_neigh1, b1r, W_self2, W_neigh2, b2r)[0]

    yv = y.reshape(2, B, N, C, L)
    out = jnp.transpose(yv, (1, 3, 0, 2, 4))
    return out.reshape(B, 2 * C, N, L).astype(jnp.float32)
